# baseline (speedup 1.0000x reference)
import functools

import jax
import jax.numpy as jnp
from jax import lax
from jax.experimental import pallas as pl
from jax.experimental.pallas import tpu as pltpu
from jax.experimental.pallas import tpu_sc as plsc

_NUM_NODES = 100000
_NUM_PINS = 400000
_NUM_NETS = 50000
_NUM_EDGES = 400000
_R_UNIT = 0.8
_C_UNIT = 0.2
_IGNORE = 100

_NC = 2
_NS = 16
_NW = _NC * _NS
_BLK = 1600
_NBLK = _NUM_EDGES // _BLK
_KMAX = (_NBLK + _NW - 1) // _NW
_LANES = 16
_VPB = _BLK // _LANES
_UNROLL = 2
_NBS_PAD = _NUM_NETS + 8
_BS_ITERS = 16


def _rc_body(posx_hbm, posy_hbm, caps_hbm, p2n_hbm, bu_hbm, bv_hbm, nbs_hbm,
             out_hbm, nbs_v,
             bu0, bu1, bv0, bv1, nu0, nu1, nv0, nv1,
             xu0, xu1, yu0, yu1, xv0, xv1, yv0, yv1, cv0, cv1,
             out0, out1,
             semf0, semf1, sems0, sems1, semo0, semo1):
    wid = lax.axis_index("s") * _NC + lax.axis_index("c")
    pltpu.sync_copy(nbs_hbm, nbs_v)

    nblk = (_NBLK - wid + _NW - 1) // _NW

    bu = (bu0, bu1)
    bv = (bv0, bv1)
    nu = (nu0, nu1)
    nv = (nv0, nv1)
    xu = (xu0, xu1)
    yu = (yu0, yu1)
    xv = (xv0, xv1)
    yv = (yv0, yv1)
    cv = (cv0, cv1)
    out = (out0, out1)
    semf = (semf0, semf1)
    sems = (sems0, sems1)
    semo = (semo0, semo1)

    def base(k):
        return (wid + k * _NW) * _BLK

    def linear(k):
        p = k % 2
        pltpu.sync_copy(bu_hbm.at[pl.ds(base(k), _BLK)], bu[p])
        pltpu.sync_copy(bv_hbm.at[pl.ds(base(k), _BLK)], bv[p])

    def first_copies(k):
        p = k % 2
        return ((p2n_hbm.at[bu[p]], nu[p], semf[p]),
                (p2n_hbm.at[bv[p]], nv[p], semf[p]))

    def second_copies(k):
        p = k % 2
        return ((posx_hbm.at[nu[p]], xu[p], sems[p]),
                (posy_hbm.at[nu[p]], yu[p], sems[p]),
                (posx_hbm.at[nv[p]], xv[p], sems[p]),
                (posy_hbm.at[nv[p]], yv[p], sems[p]),
                (caps_hbm.at[bv[p]], cv[p], sems[p]))

    def fire(copies):
        for src, dst, sem in copies:
            pltpu.async_copy(src, dst, sem)

    def drain(copies):
        for src, dst, sem in copies:
            pltpu.make_async_copy(src, dst, sem).wait()

    def out_copy(k):
        p = k % 2
        return (out[p], out_hbm.at[pl.ds(2 * base(k), 2 * _BLK)], semo[p])

    def compute(k):
        p = k % 2
        xu_v, yu_v, xv_v, yv_v, cv_v, out_v = (
            xu[p], yu[p], xv[p], yv[p], cv[p], out[p])
        bk = base(k)
        iota = lax.iota(jnp.int32, _LANES)
        lo0 = jnp.zeros((_LANES,), jnp.int32)
        hi0 = jnp.full((_LANES,), _NUM_NETS, jnp.int32)

        def vec_body(j, vcarry):
            for t in range(_UNROLL):
                off = (j * _UNROLL + t) * _LANES
                eid = bk + off + iota
                xuv = xu_v[pl.ds(off, _LANES)]
                yuv = yu_v[pl.ds(off, _LANES)]
                xvv = xv_v[pl.ds(off, _LANES)]
                yvv = yv_v[pl.ds(off, _LANES)]
                cvv = cv_v[pl.ds(off, _LANES)]
                wl = jnp.abs(xuv - xvv) + jnp.abs(yuv - yvv)

                lo, hi = lo0, hi0
                for _ in range(_BS_ITERS):
                    mid = (lo + hi) // 2
                    m = plsc.load_gather(nbs_v, [mid])
                    sel = m <= eid
                    lo = jnp.where(sel, mid, lo)
                    hi = jnp.where(sel, hi, mid)
                s0 = plsc.load_gather(nbs_v, [lo])
                s1 = plsc.load_gather(nbs_v, [lo + 1])
                deg = s1 - s0 + 1
                keep = jnp.where(deg <= _IGNORE, jnp.float32(1.0),
                                 jnp.float32(0.0))
                res = (_R_UNIT * wl) * keep
                cap = (_C_UNIT * wl + cvv) * keep
                li = off + iota
                plsc.store_scatter(out_v, [2 * li], res)
                plsc.store_scatter(out_v, [2 * li + 1], cap)
            return vcarry

        lax.fori_loop(0, _VPB // _UNROLL, vec_body, 0)

    linear(0)
    fire(first_copies(0))
    linear(1)
    fire(first_copies(1))
    drain(first_copies(0))
    fire(second_copies(0))

    for k in range(_KMAX):
        if k + 1 < _KMAX:
            @pl.when(k + 1 < nblk)
            def _(k=k):
                drain(first_copies(k + 1))
                fire(second_copies(k + 1))

        @pl.when(k < nblk)
        def _(k=k):
            drain(second_copies(k))

        if k + 2 < _KMAX:
            @pl.when(k + 2 < nblk)
            def _(k=k):
                linear(k + 2)
                fire(first_copies(k + 2))

        @pl.when(k < nblk)
        def _(k=k):
            if k >= 2:
                drain((out_copy(k - 2),))
            compute(k)
            fire((out_copy(k),))

    for j in (_KMAX - 2, _KMAX - 1):
        @pl.when(j < nblk)
        def _(j=j):
            drain((out_copy(j),))


@functools.lru_cache(maxsize=1)
def _build():
    mesh = plsc.VectorSubcoreMesh(core_axis_name="c", subcore_axis_name="s")
    ivec = pltpu.VMEM((_BLK,), jnp.int32)
    fvec = pltpu.VMEM((_BLK,), jnp.float32)
    return pl.kernel(
        _rc_body,
        out_type=jax.ShapeDtypeStruct((2 * _NUM_EDGES,), jnp.float32),
        mesh=mesh,
        compiler_params=pltpu.CompilerParams(needs_layout_passes=False),
        scratch_types=[
            pltpu.VMEM((_NBS_PAD,), jnp.int32),
            ivec, ivec, ivec, ivec,
            ivec, ivec, ivec, ivec,
            fvec, fvec, fvec, fvec,
            fvec, fvec, fvec, fvec,
            fvec, fvec,
            pltpu.VMEM((2 * _BLK,), jnp.float32),
            pltpu.VMEM((2 * _BLK,), jnp.float32),
            pltpu.SemaphoreType.DMA, pltpu.SemaphoreType.DMA,
            pltpu.SemaphoreType.DMA, pltpu.SemaphoreType.DMA,
            pltpu.SemaphoreType.DMA, pltpu.SemaphoreType.DMA,
        ],
    )


def kernel(pos, pin_caps, pin2node_map, branch_u, branch_v, net_branch_start,
           driver_pin_indices):
    posx = pos[:, 0]
    posy = pos[:, 1]
    nbs = jnp.concatenate(
        [net_branch_start,
         jnp.full((_NBS_PAD - _NUM_NETS - 1,), _NUM_EDGES, jnp.int32)])
    out = _build()(posx, posy, pin_caps, pin2node_map, branch_u, branch_v,
                   nbs)
    return out.reshape(_NUM_EDGES, 2)
